# baseline (speedup 1.0000x reference)
import functools

import jax
import jax.numpy as jnp
from jax.experimental import pallas as pl
from jax.experimental.pallas import tpu as pltpu

_TOKEN_OFF = 4000
_TIME_ROWS = 64


def _hybrid_tile_kernel(TILE, TG, V_PAD, ids_ref, idcol_ref, tcol_ref,
                        tab3d_ref, tab2d_ref, ttab_ref, pe_ref, out_ref):
    TM = TILE - TG
    base = pl.program_id(0) * TILE

    if TM:
        lane = jax.lax.broadcasted_iota(jnp.int32, (TM, V_PAD), 1)
        oh = (idcol_ref[TG:] == lane).astype(jnp.float32)
        mm = jnp.dot(oh, tab2d_ref[...], preferred_element_type=jnp.float32)
        out_ref[TG:] = mm

    for mi in range(TG):
        out_ref[mi] = tab3d_ref[ids_ref[base + mi], 0]

    tlane = jax.lax.broadcasted_iota(jnp.int32, (TILE, _TIME_ROWS), 1)
    t_oh = (tcol_ref[...] == tlane).astype(jnp.float32)
    tmm = jnp.dot(t_oh, ttab_ref[...], preferred_element_type=jnp.float32)
    out_ref[...] = out_ref[...] + (tmm + pe_ref[...])


def kernel(x, time, fused_table, pe_scaled):
    B, S = x.shape
    v_pad, d_model = fused_table.shape

    ids = x.astype(jnp.int32).reshape(B * S)
    idcol = ids.reshape(B * S, 1)
    tcol = time.astype(jnp.int32).reshape(B * S, 1)
    tab3d = fused_table.reshape(v_pad, 1, d_model)
    ttab = fused_table[_TOKEN_OFF:_TOKEN_OFF + _TIME_ROWS]
    pe = pe_scaled[:S]

    tile = 2048
    while S % tile:
        tile //= 2
    n_s = S // tile
    grid = (B * n_s,)
    tg = min(tile, max(0, int(tile * 0.59) & ~7))

    body = functools.partial(_hybrid_tile_kernel, tile, tg, v_pad)

    m_total = B * S
    tm = tile - tg
    bytes_accessed = (3 * m_total * 4
                      + 2 * v_pad * d_model * 4
                      + S * d_model * 4
                      + m_total * d_model * 4)
    flops = 2 * (tm * n_s) * v_pad * d_model + 3 * m_total * d_model
    cost = pl.CostEstimate(flops=flops, transcendentals=0,
                           bytes_accessed=bytes_accessed)

    out = pl.pallas_call(
        body,
        out_shape=jax.ShapeDtypeStruct((B * S, d_model), jnp.float32),
        grid=grid,
        in_specs=[
            pl.BlockSpec(memory_space=pltpu.SMEM),
            pl.BlockSpec((tile, 1), lambda i: (i, 0)),
            pl.BlockSpec((tile, 1), lambda i: (i, 0)),
            pl.BlockSpec((v_pad, 1, d_model), lambda i: (0, 0, 0)),
            pl.BlockSpec((v_pad, d_model), lambda i: (0, 0)),
            pl.BlockSpec((_TIME_ROWS, d_model), lambda i: (0, 0)),
            pl.BlockSpec((tile, d_model), lambda i: (i % n_s, 0)),
        ],
        out_specs=pl.BlockSpec((tile, d_model), lambda i: (i, 0)),
        compiler_params=pltpu.CompilerParams(
            dimension_semantics=("parallel",)),
        cost_estimate=cost,
    )(ids, idcol, tcol, tab3d, fused_table, ttab, pe)
    return out.reshape(B, S, d_model)


# speedup vs baseline: 1.4756x; 1.4756x over previous
import functools

import jax
import jax.numpy as jnp
from jax.experimental import pallas as pl
from jax.experimental.pallas import tpu as pltpu

_TOKEN_OFF = 4000
_TIME_ROWS = 64


def _gather_tile_kernel(TILE, ids_ref, tcol_ref, tab_ref, ttab_ref, pe_ref,
                        out_ref):
    base = pl.program_id(0) * TILE
    for mi in range(TILE):
        out_ref[mi] = tab_ref[ids_ref[base + mi], 0]
    lane = jax.lax.broadcasted_iota(jnp.int32, (TILE, _TIME_ROWS), 1)
    t_oh = (tcol_ref[...] == lane).astype(jnp.float32)
    tmm = jnp.dot(t_oh, ttab_ref[...], preferred_element_type=jnp.float32)
    out_ref[...] = out_ref[...] + (tmm + pe_ref[...])


def kernel(x, time, fused_table, pe_scaled):
    B, S = x.shape
    v_pad, d_model = fused_table.shape

    ids = x.astype(jnp.int32).reshape(B * S)
    tcol = time.astype(jnp.int32).reshape(B * S, 1)
    tab3d = fused_table.reshape(v_pad, 1, d_model)
    ttab = fused_table[_TOKEN_OFF:_TOKEN_OFF + _TIME_ROWS]
    pe = pe_scaled[:S]

    tile = 4096
    while S % tile:
        tile //= 2
    n_s = S // tile
    grid = (B * n_s,)

    body = functools.partial(_gather_tile_kernel, tile)

    m_total = B * S
    bytes_accessed = (2 * m_total * 4
                      + v_pad * d_model * 4
                      + S * d_model * 4
                      + m_total * d_model * 4)
    cost = pl.CostEstimate(flops=3 * m_total * d_model, transcendentals=0,
                           bytes_accessed=bytes_accessed)

    out = pl.pallas_call(
        body,
        out_shape=jax.ShapeDtypeStruct((B * S, d_model), jnp.float32),
        grid=grid,
        in_specs=[
            pl.BlockSpec(memory_space=pltpu.SMEM),
            pl.BlockSpec((tile, 1), lambda i: (i, 0)),
            pl.BlockSpec((v_pad, 1, d_model), lambda i: (0, 0, 0)),
            pl.BlockSpec((_TIME_ROWS, d_model), lambda i: (0, 0)),
            pl.BlockSpec((tile, d_model), lambda i: (i % n_s, 0)),
        ],
        out_specs=pl.BlockSpec((tile, d_model), lambda i: (i, 0)),
        compiler_params=pltpu.CompilerParams(
            dimension_semantics=("parallel",)),
        cost_estimate=cost,
    )(ids, tcol, tab3d, ttab, pe)
    return out.reshape(B, S, d_model)


# speedup vs baseline: 1.8073x; 1.2247x over previous
import functools

import jax
import jax.numpy as jnp
from jax.experimental import pallas as pl
from jax.experimental.pallas import tpu as pltpu

_TOKEN_OFF = 4000
_TIME_ROWS = 64


def _gather_tile_kernel(TILE, ids_ref, trow_ref, tab_ref, ttab_ref, pe_ref,
                        out_ref):
    base = pl.program_id(0) * TILE
    for mi in range(TILE):
        out_ref[mi] = tab_ref[ids_ref[base + mi], 0]
    row = jax.lax.broadcasted_iota(jnp.int32, (_TIME_ROWS, TILE), 0)
    t_oh = (trow_ref[0] == row).astype(jnp.float32)
    tmm = jax.lax.dot_general(t_oh, ttab_ref[...],
                              (((0,), (0,)), ((), ())),
                              preferred_element_type=jnp.float32)
    out_ref[...] = out_ref[...] + (tmm + pe_ref[...])


def kernel(x, time, fused_table, pe_scaled):
    B, S = x.shape
    v_pad, d_model = fused_table.shape

    ids = x.astype(jnp.int32).reshape(B * S)
    tab3d = fused_table.reshape(v_pad, 1, d_model)
    ttab = fused_table[_TOKEN_OFF:_TOKEN_OFF + _TIME_ROWS]
    pe = pe_scaled[:S]

    tile = 4096
    while S % tile:
        tile //= 2
    n_s = S // tile
    grid = (B * n_s,)
    trow = time.astype(jnp.int32).reshape(B * n_s, 1, tile)

    body = functools.partial(_gather_tile_kernel, tile)

    m_total = B * S
    bytes_accessed = (2 * m_total * 4
                      + v_pad * d_model * 4
                      + S * d_model * 4
                      + m_total * d_model * 4)
    cost = pl.CostEstimate(flops=3 * m_total * d_model, transcendentals=0,
                           bytes_accessed=bytes_accessed)

    out = pl.pallas_call(
        body,
        out_shape=jax.ShapeDtypeStruct((B * S, d_model), jnp.float32),
        grid=grid,
        in_specs=[
            pl.BlockSpec(memory_space=pltpu.SMEM),
            pl.BlockSpec((1, 1, tile), lambda i: (i, 0, 0)),
            pl.BlockSpec((v_pad, 1, d_model), lambda i: (0, 0, 0)),
            pl.BlockSpec((_TIME_ROWS, d_model), lambda i: (0, 0)),
            pl.BlockSpec((tile, d_model), lambda i: (i % n_s, 0)),
        ],
        out_specs=pl.BlockSpec((tile, d_model), lambda i: (i, 0)),
        compiler_params=pltpu.CompilerParams(
            dimension_semantics=("parallel",)),
        cost_estimate=cost,
    )(ids, trow, tab3d, ttab, pe)
    return out.reshape(B, S, d_model)


# speedup vs baseline: 1.8173x; 1.0055x over previous
import functools

import jax
import jax.numpy as jnp
from jax.experimental import pallas as pl
from jax.experimental.pallas import tpu as pltpu

_TOKEN_OFF = 4000
_TIME_ROWS = 64


def _gather_tile_kernel(TILE, ids_ref, trow_ref, tab_ref, ttab_ref, pe_ref,
                        out_ref):
    base = pl.program_id(0) * TILE
    sub = ids_ref.at[pl.ds(base, TILE)]
    for mi in range(TILE):
        out_ref[mi] = tab_ref[sub[mi], 0]
    row = jax.lax.broadcasted_iota(jnp.int32, (_TIME_ROWS, TILE), 0)
    t_oh = (trow_ref[0] == row).astype(jnp.float32)
    tmm = jax.lax.dot_general(t_oh, ttab_ref[...],
                              (((0,), (0,)), ((), ())),
                              preferred_element_type=jnp.float32)
    out_ref[...] = out_ref[...] + (tmm + pe_ref[...])


def kernel(x, time, fused_table, pe_scaled):
    B, S = x.shape
    v_pad, d_model = fused_table.shape

    ids = x.astype(jnp.int32).reshape(B * S)
    tab3d = fused_table.reshape(v_pad, 1, d_model)
    ttab = fused_table[_TOKEN_OFF:_TOKEN_OFF + _TIME_ROWS]
    pe = pe_scaled[:S]

    tile = 4096
    while S % tile:
        tile //= 2
    n_s = S // tile
    grid = (B * n_s,)
    trow = time.astype(jnp.int32).reshape(B * n_s, 1, tile)

    body = functools.partial(_gather_tile_kernel, tile)

    m_total = B * S
    bytes_accessed = (2 * m_total * 4
                      + v_pad * d_model * 4
                      + S * d_model * 4
                      + m_total * d_model * 4)
    cost = pl.CostEstimate(flops=3 * m_total * d_model, transcendentals=0,
                           bytes_accessed=bytes_accessed)

    out = pl.pallas_call(
        body,
        out_shape=jax.ShapeDtypeStruct((B * S, d_model), jnp.float32),
        grid=grid,
        in_specs=[
            pl.BlockSpec(memory_space=pltpu.SMEM),
            pl.BlockSpec((1, 1, tile), lambda i: (i, 0, 0)),
            pl.BlockSpec((v_pad, 1, d_model), lambda i: (0, 0, 0)),
            pl.BlockSpec((_TIME_ROWS, d_model), lambda i: (0, 0)),
            pl.BlockSpec((tile, d_model), lambda i: (i % n_s, 0)),
        ],
        out_specs=pl.BlockSpec((tile, d_model), lambda i: (i, 0)),
        compiler_params=pltpu.CompilerParams(
            dimension_semantics=("parallel",)),
        cost_estimate=cost,
    )(ids, trow, tab3d, ttab, pe)
    return out.reshape(B, S, d_model)


# speedup vs baseline: 2.1470x; 1.1815x over previous
import functools

import jax
import jax.numpy as jnp
from jax.experimental import pallas as pl
from jax.experimental.pallas import tpu as pltpu

_TOKEN_OFF = 4000
_TIME_ROWS = 64


def _gather_tile_kernel(TILE, ids_ref, trow_ref, tab_ref, ttab_ref, pe_ref,
                        out_ref):
    base = pl.program_id(0) * TILE
    sub = ids_ref.at[pl.ds(base, TILE)]
    for k in range(TILE // 8):
        rows = [tab_ref[sub[8 * k + j], 0] for j in range(8)]
        out_ref[8 * k:8 * k + 8] = jnp.stack(rows)
    row = jax.lax.broadcasted_iota(jnp.int32, (_TIME_ROWS, TILE), 0)
    t_oh = (trow_ref[0] == row).astype(jnp.float32)
    tmm = jax.lax.dot_general(t_oh, ttab_ref[...],
                              (((0,), (0,)), ((), ())),
                              preferred_element_type=jnp.float32)
    out_ref[...] = out_ref[...] + (tmm + pe_ref[...])


def kernel(x, time, fused_table, pe_scaled):
    B, S = x.shape
    v_pad, d_model = fused_table.shape

    ids = x.astype(jnp.int32).reshape(B * S)
    tab3d = fused_table.reshape(v_pad, 1, d_model)
    ttab = fused_table[_TOKEN_OFF:_TOKEN_OFF + _TIME_ROWS]
    pe = pe_scaled[:S]

    tile = 4096
    while S % tile:
        tile //= 2
    n_s = S // tile
    grid = (B * n_s,)
    trow = time.astype(jnp.int32).reshape(B * n_s, 1, tile)

    body = functools.partial(_gather_tile_kernel, tile)

    m_total = B * S
    bytes_accessed = (2 * m_total * 4
                      + v_pad * d_model * 4
                      + S * d_model * 4
                      + m_total * d_model * 4)
    cost = pl.CostEstimate(flops=3 * m_total * d_model, transcendentals=0,
                           bytes_accessed=bytes_accessed)

    out = pl.pallas_call(
        body,
        out_shape=jax.ShapeDtypeStruct((B * S, d_model), jnp.float32),
        grid=grid,
        in_specs=[
            pl.BlockSpec(memory_space=pltpu.SMEM),
            pl.BlockSpec((1, 1, tile), lambda i: (i, 0, 0)),
            pl.BlockSpec((v_pad, 1, d_model), lambda i: (0, 0, 0)),
            pl.BlockSpec((_TIME_ROWS, d_model), lambda i: (0, 0)),
            pl.BlockSpec((tile, d_model), lambda i: (i % n_s, 0)),
        ],
        out_specs=pl.BlockSpec((tile, d_model), lambda i: (i, 0)),
        compiler_params=pltpu.CompilerParams(
            dimension_semantics=("parallel",)),
        cost_estimate=cost,
    )(ids, trow, tab3d, ttab, pe)
    return out.reshape(B, S, d_model)


# speedup vs baseline: 2.2977x; 1.0702x over previous
import functools

import jax
import jax.numpy as jnp
from jax.experimental import pallas as pl
from jax.experimental.pallas import tpu as pltpu

_TOKEN_OFF = 4000
_TIME_ROWS = 64


def _gather_tile_kernel(TILE, ids_ref, trow_ref, tab_ref, ttab_ref, pe_ref,
                        out_ref, sscr_ref, sem_ref):
    cp = pltpu.make_async_copy(ids_ref, sscr_ref, sem_ref)
    cp.start()
    cp.wait()
    for k in range(TILE // 8):
        rows = [tab_ref[sscr_ref[0, 0, 8 * k + j], 0] for j in range(8)]
        out_ref[8 * k:8 * k + 8] = jnp.stack(rows)
    row = jax.lax.broadcasted_iota(jnp.int32, (_TIME_ROWS, TILE), 0)
    t_oh = (trow_ref[0] == row).astype(jnp.float32)
    tmm = jax.lax.dot_general(t_oh, ttab_ref[...],
                              (((0,), (0,)), ((), ())),
                              preferred_element_type=jnp.float32)
    out_ref[...] = out_ref[...] + (tmm + pe_ref[...])


def kernel(x, time, fused_table, pe_scaled):
    B, S = x.shape
    v_pad, d_model = fused_table.shape

    ids = x.astype(jnp.int32)
    tab3d = fused_table.reshape(v_pad, 1, d_model)
    ttab = fused_table[_TOKEN_OFF:_TOKEN_OFF + _TIME_ROWS]
    pe = pe_scaled[:S]

    tile = 4096
    while S % tile:
        tile //= 2
    n_s = S // tile
    grid = (B * n_s,)
    trow = time.astype(jnp.int32).reshape(B * n_s, 1, tile)
    idsb = ids.reshape(B * n_s, 1, tile)

    body = functools.partial(_gather_tile_kernel, tile)

    m_total = B * S
    bytes_accessed = (2 * m_total * 4
                      + v_pad * d_model * 4
                      + S * d_model * 4
                      + m_total * d_model * 4)
    cost = pl.CostEstimate(flops=3 * m_total * d_model, transcendentals=0,
                           bytes_accessed=bytes_accessed)

    out = pl.pallas_call(
        body,
        out_shape=jax.ShapeDtypeStruct((B * S, d_model), jnp.float32),
        grid=grid,
        in_specs=[
            pl.BlockSpec((1, 1, tile), lambda i: (i, 0, 0)),
            pl.BlockSpec((1, 1, tile), lambda i: (i, 0, 0)),
            pl.BlockSpec((v_pad, 1, d_model), lambda i: (0, 0, 0)),
            pl.BlockSpec((_TIME_ROWS, d_model), lambda i: (0, 0)),
            pl.BlockSpec((tile, d_model), lambda i: (i % n_s, 0)),
        ],
        out_specs=pl.BlockSpec((tile, d_model), lambda i: (i, 0)),
        scratch_shapes=[pltpu.SMEM((1, 1, tile), jnp.int32),
                        pltpu.SemaphoreType.DMA],
        compiler_params=pltpu.CompilerParams(
            dimension_semantics=("parallel",)),
        cost_estimate=cost,
    )(idsb, trow, tab3d, ttab, pe)
    return out.reshape(B, S, d_model)


# speedup vs baseline: 2.3267x; 1.0126x over previous
import functools

import jax
import jax.numpy as jnp
from jax.experimental import pallas as pl
from jax.experimental.pallas import tpu as pltpu

_TOKEN_OFF = 4000
_TIME_ROWS = 64


def _gather_tile_kernel(TILE, ids_ref, trow_ref, tab_ref, ttab_ref, pe_ref,
                        out_ref, sscr_ref, pescr_ref, sem_ref):
    cp = pltpu.make_async_copy(ids_ref, sscr_ref, sem_ref)
    cp.start()
    row = jax.lax.broadcasted_iota(jnp.int32, (_TIME_ROWS, TILE), 0)
    t_oh = (trow_ref[0] == row).astype(jnp.float32)
    tmm = jax.lax.dot_general(t_oh, ttab_ref[...],
                              (((0,), (0,)), ((), ())),
                              preferred_element_type=jnp.float32)
    pescr_ref[...] = tmm + pe_ref[...]
    cp.wait()
    for k in range(TILE // 8):
        rows = [tab_ref[sscr_ref[0, 0, 8 * k + j], 0] for j in range(8)]
        out_ref[8 * k:8 * k + 8] = jnp.stack(rows)
    out_ref[...] = out_ref[...] + pescr_ref[...]


def kernel(x, time, fused_table, pe_scaled):
    B, S = x.shape
    v_pad, d_model = fused_table.shape

    ids = x.astype(jnp.int32)
    tab3d = fused_table.reshape(v_pad, 1, d_model)
    ttab = fused_table[_TOKEN_OFF:_TOKEN_OFF + _TIME_ROWS]
    pe = pe_scaled[:S]

    tile = 4096
    while S % tile:
        tile //= 2
    n_s = S // tile
    grid = (B * n_s,)
    trow = time.astype(jnp.int32).reshape(B * n_s, 1, tile)
    idsb = ids.reshape(B * n_s, 1, tile)

    body = functools.partial(_gather_tile_kernel, tile)

    m_total = B * S
    bytes_accessed = (2 * m_total * 4
                      + v_pad * d_model * 4
                      + S * d_model * 4
                      + m_total * d_model * 4)
    cost = pl.CostEstimate(flops=3 * m_total * d_model, transcendentals=0,
                           bytes_accessed=bytes_accessed)

    out = pl.pallas_call(
        body,
        out_shape=jax.ShapeDtypeStruct((B * S, d_model), jnp.float32),
        grid=grid,
        in_specs=[
            pl.BlockSpec((1, 1, tile), lambda i: (i, 0, 0)),
            pl.BlockSpec((1, 1, tile), lambda i: (i, 0, 0)),
            pl.BlockSpec((v_pad, 1, d_model), lambda i: (0, 0, 0)),
            pl.BlockSpec((_TIME_ROWS, d_model), lambda i: (0, 0)),
            pl.BlockSpec((tile, d_model), lambda i: (i % n_s, 0)),
        ],
        out_specs=pl.BlockSpec((tile, d_model), lambda i: (i, 0)),
        scratch_shapes=[pltpu.SMEM((1, 1, tile), jnp.int32),
                        pltpu.VMEM((tile, d_model), jnp.float32),
                        pltpu.SemaphoreType.DMA],
        compiler_params=pltpu.CompilerParams(
            dimension_semantics=("parallel",)),
        cost_estimate=cost,
    )(idsb, trow, tab3d, ttab, pe)
    return out.reshape(B, S, d_model)


# speedup vs baseline: 2.4167x; 1.0387x over previous
import functools

import jax
import jax.numpy as jnp
from jax.experimental import pallas as pl
from jax.experimental.pallas import tpu as pltpu

_TOKEN_OFF = 4000
_TIME_ROWS = 64


def _gather_tile_kernel(TILE, ids_ref, trow_ref, tab_ref, ttab_ref, pe_ref,
                        out_ref, sscr_ref, pescr_ref, sem_ref):
    cp = pltpu.make_async_copy(ids_ref, sscr_ref, sem_ref)
    cp.start()
    row = jax.lax.broadcasted_iota(jnp.int32, (_TIME_ROWS, TILE), 0)
    t_oh = (trow_ref[0] == row).astype(jnp.float32)
    tmm = jax.lax.dot_general(t_oh, ttab_ref[...],
                              (((0,), (0,)), ((), ())),
                              preferred_element_type=jnp.float32)
    pescr_ref[...] = tmm + pe_ref[...]
    cp.wait()
    for k in range(TILE // 8):
        rows = [tab_ref[sscr_ref[0, 0, 8 * k + j], 0] for j in range(8)]
        out_ref[8 * k:8 * k + 8] = jnp.stack(rows) + pescr_ref[8 * k:8 * k + 8]


def kernel(x, time, fused_table, pe_scaled):
    B, S = x.shape
    v_pad, d_model = fused_table.shape

    ids = x.astype(jnp.int32)
    tab3d = fused_table.reshape(v_pad, 1, d_model)
    ttab = fused_table[_TOKEN_OFF:_TOKEN_OFF + _TIME_ROWS]
    pe = pe_scaled[:S]

    tile = 4096
    while S % tile:
        tile //= 2
    n_s = S // tile
    grid = (B * n_s,)
    trow = time.astype(jnp.int32).reshape(B * n_s, 1, tile)
    idsb = ids.reshape(B * n_s, 1, tile)

    body = functools.partial(_gather_tile_kernel, tile)

    m_total = B * S
    bytes_accessed = (2 * m_total * 4
                      + v_pad * d_model * 4
                      + S * d_model * 4
                      + m_total * d_model * 4)
    cost = pl.CostEstimate(flops=3 * m_total * d_model, transcendentals=0,
                           bytes_accessed=bytes_accessed)

    out = pl.pallas_call(
        body,
        out_shape=jax.ShapeDtypeStruct((B * S, d_model), jnp.float32),
        grid=grid,
        in_specs=[
            pl.BlockSpec((1, 1, tile), lambda i: (i, 0, 0)),
            pl.BlockSpec((1, 1, tile), lambda i: (i, 0, 0)),
            pl.BlockSpec((v_pad, 1, d_model), lambda i: (0, 0, 0)),
            pl.BlockSpec((_TIME_ROWS, d_model), lambda i: (0, 0)),
            pl.BlockSpec((tile, d_model), lambda i: (i % n_s, 0)),
        ],
        out_specs=pl.BlockSpec((tile, d_model), lambda i: (i, 0)),
        scratch_shapes=[pltpu.SMEM((1, 1, tile), jnp.int32),
                        pltpu.VMEM((tile, d_model), jnp.float32),
                        pltpu.SemaphoreType.DMA],
        compiler_params=pltpu.CompilerParams(
            dimension_semantics=("parallel",)),
        cost_estimate=cost,
    )(idsb, trow, tab3d, ttab, pe)
    return out.reshape(B, S, d_model)


# speedup vs baseline: 2.4978x; 1.0336x over previous
import functools

import jax
import jax.numpy as jnp
from jax.experimental import pallas as pl
from jax.experimental.pallas import tpu as pltpu

_TOKEN_OFF = 4000
_TIME_ROWS = 64


def _gather_tile_kernel(TILE, ids_ref, trow_ref, tab_ref, ttab_ref, pe_ref,
                        out_ref, sscr_ref, pescr_ref, sem_ref):
    cp = pltpu.make_async_copy(ids_ref, sscr_ref, sem_ref)
    cp.start()
    row = jax.lax.broadcasted_iota(jnp.int32, (_TIME_ROWS, TILE), 0)
    t_oh = (trow_ref[0] == row).astype(jnp.float32)
    tmm = jax.lax.dot_general(t_oh, ttab_ref[...],
                              (((0,), (0,)), ((), ())),
                              preferred_element_type=jnp.float32)
    pescr_ref[...] = tmm + pe_ref[...]
    cp.wait()
    for k in range(TILE // 8):
        rows = [tab_ref[sscr_ref[0, 0, 8 * k + j], 0] for j in range(8)]
        out_ref[8 * k:8 * k + 8] = jnp.stack(rows) + pescr_ref[8 * k:8 * k + 8]


def kernel(x, time, fused_table, pe_scaled):
    B, S = x.shape
    v_pad, d_model = fused_table.shape

    ids = x.astype(jnp.int32)
    tab3d = fused_table.reshape(v_pad, 1, d_model)
    ttab = fused_table[_TOKEN_OFF:_TOKEN_OFF + _TIME_ROWS]
    pe = pe_scaled[:S]

    tile = 8192
    while S % tile:
        tile //= 2
    n_s = S // tile
    grid = (B * n_s,)
    trow = time.astype(jnp.int32).reshape(B * n_s, 1, tile)
    idsb = ids.reshape(B * n_s, 1, tile)

    body = functools.partial(_gather_tile_kernel, tile)

    m_total = B * S
    bytes_accessed = (2 * m_total * 4
                      + v_pad * d_model * 4
                      + S * d_model * 4
                      + m_total * d_model * 4)
    cost = pl.CostEstimate(flops=3 * m_total * d_model, transcendentals=0,
                           bytes_accessed=bytes_accessed)

    out = pl.pallas_call(
        body,
        out_shape=jax.ShapeDtypeStruct((B * S, d_model), jnp.float32),
        grid=grid,
        in_specs=[
            pl.BlockSpec((1, 1, tile), lambda i: (i, 0, 0)),
            pl.BlockSpec((1, 1, tile), lambda i: (i, 0, 0)),
            pl.BlockSpec((v_pad, 1, d_model), lambda i: (0, 0, 0)),
            pl.BlockSpec((_TIME_ROWS, d_model), lambda i: (0, 0)),
            pl.BlockSpec((tile, d_model), lambda i: (i % n_s, 0)),
        ],
        out_specs=pl.BlockSpec((tile, d_model), lambda i: (i, 0)),
        scratch_shapes=[pltpu.SMEM((1, 1, tile), jnp.int32),
                        pltpu.VMEM((tile, d_model), jnp.float32),
                        pltpu.SemaphoreType.DMA],
        compiler_params=pltpu.CompilerParams(
            dimension_semantics=("parallel",)),
        cost_estimate=cost,
    )(idsb, trow, tab3d, ttab, pe)
    return out.reshape(B, S, d_model)
